# baseline (speedup 1.0000x reference)
import functools

import jax
import jax.numpy as jnp
import numpy as np
from jax import lax
from jax.experimental import pallas as pl
from jax.experimental.pallas import tpu as pltpu
from jax.experimental.pallas import tpu_sc as plsc

_CLIP = np.array([(1, 0, 1), (2, 0, 1), (3, 0, -1), (4, 0, -1), (5, 0, -1),
                  (6, 0, -1), (7, 0, -1), (8, 0, -1), (9, 0, -1), (12, 0, -1),
                  (13, 1, 1), (14, 1, -1), (16, 1, 1), (17, 1, -1),
                  (18, 1, 1), (19, 1, -1)], dtype=np.int64)
_ZERO = np.array([(10, 0), (10, 1), (10, 2), (11, 0), (11, 1), (11, 2),
                  (15, 0), (15, 1), (15, 2), (20, 1), (21, 1)], dtype=np.int64)

_N_ROWS = 16384
_N_COLS = 63
_N_TERMS = len(_CLIP) + len(_ZERO)
_PAT = 1008
_NW = 32
_ROWS_PER_W = _N_ROWS // _NW
_WORDS_PER_W = _ROWS_PER_W * _N_COLS
_BLOCKS = _WORDS_PER_W // _PAT


def _weights():
    sa = np.zeros(_N_COLS, np.float64)
    sb = np.zeros(_N_COLS, np.float64)
    for j, a, s in _CLIP:
        c = (j - 1) * 3 + a
        if s > 0:
            sa[c] = 1.0
        else:
            sa[c] = -1.0
    for j, a in _ZERO:
        c = (j - 1) * 3 + a
        sa[c] = 1.0
        sb[c] = -1.0
    scale = 1.0 / (_N_ROWS * _N_TERMS)
    sa *= scale
    sb *= scale
    reps = _PAT // _N_COLS
    return (np.tile(sa, reps).astype(np.float32),
            np.tile(sb, reps).astype(np.float32))


_SA, _SB = _weights()

_mesh = plsc.VectorSubcoreMesh(core_axis_name="c", subcore_axis_name="s")


@functools.partial(
    pl.kernel,
    out_type=jax.ShapeDtypeStruct((_NW * 16,), jnp.float32),
    mesh=_mesh,
    scratch_types=[
        pltpu.VMEM((_WORDS_PER_W,), jnp.float32),
        pltpu.VMEM((_PAT,), jnp.float32),
        pltpu.VMEM((_PAT,), jnp.float32),
        pltpu.VMEM((16,), jnp.float32),
    ],
)
def _sc_partial_sums(pose_hbm, sa_hbm, sb_hbm, out_hbm, x_v, sa_v, sb_v,
                     acc_v):
    wid = lax.axis_index("s") * 2 + lax.axis_index("c")
    base = wid * _WORDS_PER_W
    pltpu.sync_copy(sa_hbm, sa_v)
    pltpu.sync_copy(sb_hbm, sb_v)
    pltpu.sync_copy(pose_hbm.at[pl.ds(base, _WORDS_PER_W)], x_v)

    acc = jnp.zeros((16,), jnp.float32)
    for j in range(_PAT // 16):
        wa = sa_v[j * 16:(j + 1) * 16]
        wb = sb_v[j * 16:(j + 1) * 16]

        def body(b, a, j=j, wa=wa, wb=wb):
            x = x_v[pl.ds(b * _PAT + j * 16, 16)]
            return a + jnp.maximum(x * wa, x * wb)

        acc = lax.fori_loop(0, _BLOCKS, body, acc)

    acc_v[...] = acc
    pltpu.sync_copy(acc_v, out_hbm.at[pl.ds(wid * 16, 16)])


def kernel(pose):
    partials = _sc_partial_sums(pose.reshape(-1), jnp.asarray(_SA),
                                jnp.asarray(_SB))
    return jnp.sum(partials)


# speedup vs baseline: 1.3988x; 1.3988x over previous
import functools

import jax
import jax.numpy as jnp
import numpy as np
from jax import lax
from jax.experimental import pallas as pl
from jax.experimental.pallas import tpu as pltpu
from jax.experimental.pallas import tpu_sc as plsc

_CLIP = np.array([(1, 0, 1), (2, 0, 1), (3, 0, -1), (4, 0, -1), (5, 0, -1),
                  (6, 0, -1), (7, 0, -1), (8, 0, -1), (9, 0, -1), (12, 0, -1),
                  (13, 1, 1), (14, 1, -1), (16, 1, 1), (17, 1, -1),
                  (18, 1, 1), (19, 1, -1)], dtype=np.int64)
_ZERO = np.array([(10, 0), (10, 1), (10, 2), (11, 0), (11, 1), (11, 2),
                  (15, 0), (15, 1), (15, 2), (20, 1), (21, 1)], dtype=np.int64)

_N_ROWS = 16384
_N_COLS = 63
_N_TERMS = len(_CLIP) + len(_ZERO)
_SCALE = 1.0 / (_N_ROWS * _N_TERMS)

_P_COLS = tuple(int((j - 1) * 3 + a) for j, a, s in _CLIP if s > 0)
_N_COLS_NEG = tuple(int((j - 1) * 3 + a) for j, a, s in _CLIP if s < 0)
_Z_COLS = tuple(int((j - 1) * 3 + a) for j, a in _ZERO)

_NW = 32
_ROWS_PER_W = _N_ROWS // _NW
_WORDS_PER_W = _ROWS_PER_W * _N_COLS
_BLK_WORDS = 16 * _N_COLS
_BLOCKS = _ROWS_PER_W // 16

_mesh = plsc.VectorSubcoreMesh(core_axis_name="c", subcore_axis_name="s")


@functools.partial(
    pl.kernel,
    out_type=jax.ShapeDtypeStruct((_NW * 16,), jnp.float32),
    mesh=_mesh,
    scratch_types=[
        pltpu.VMEM((_WORDS_PER_W,), jnp.float32),
        pltpu.VMEM((16,), jnp.float32),
    ],
    compiler_params=pltpu.CompilerParams(needs_layout_passes=False),
)
def _sc_partial_sums(pose_hbm, out_hbm, x_v, acc_v):
    wid = lax.axis_index("s") * 2 + lax.axis_index("c")
    base = wid * _WORDS_PER_W
    pltpu.sync_copy(pose_hbm.at[pl.ds(base, _WORDS_PER_W)], x_v)

    row_off = lax.iota(jnp.int32, 16) * _N_COLS
    zero = jnp.zeros((16,), jnp.float32)

    def body(b, accs):
        p0, p1, n0, n1, z0, z1 = accs
        vb = row_off + b * _BLK_WORDS
        pr = []
        for k, c in enumerate(_P_COLS):
            x = plsc.load_gather(x_v, [vb + c])
            pr.append(jnp.maximum(x, 0.0))
        nr = []
        for k, c in enumerate(_N_COLS_NEG):
            x = plsc.load_gather(x_v, [vb + c])
            nr.append(jnp.minimum(x, 0.0))
        zr = []
        for k, c in enumerate(_Z_COLS):
            x = plsc.load_gather(x_v, [vb + c])
            zr.append(jnp.abs(x))
        p0 = p0 + sum(pr[0::2], zero)
        p1 = p1 + sum(pr[1::2], zero)
        n0 = n0 + sum(nr[0::2], zero)
        n1 = n1 + sum(nr[1::2], zero)
        z0 = z0 + sum(zr[0::2], zero)
        z1 = z1 + sum(zr[1::2], zero)
        return (p0, p1, n0, n1, z0, z1)

    init = (zero,) * 6
    p0, p1, n0, n1, z0, z1 = lax.fori_loop(0, _BLOCKS, body, init)
    acc = ((p0 + p1) - (n0 + n1) + (z0 + z1)) * jnp.float32(_SCALE)
    acc_v[...] = acc
    pltpu.sync_copy(acc_v, out_hbm.at[pl.ds(wid * 16, 16)])


def kernel(pose):
    partials = _sc_partial_sums(pose.reshape(-1))
    return jnp.sum(partials)


# speedup vs baseline: 1.5636x; 1.1179x over previous
import functools

import jax
import jax.numpy as jnp
import numpy as np
from jax import lax
from jax.experimental import pallas as pl
from jax.experimental.pallas import tpu as pltpu
from jax.experimental.pallas import tpu_sc as plsc

_CLIP = np.array([(1, 0, 1), (2, 0, 1), (3, 0, -1), (4, 0, -1), (5, 0, -1),
                  (6, 0, -1), (7, 0, -1), (8, 0, -1), (9, 0, -1), (12, 0, -1),
                  (13, 1, 1), (14, 1, -1), (16, 1, 1), (17, 1, -1),
                  (18, 1, 1), (19, 1, -1)], dtype=np.int64)
_ZERO = np.array([(10, 0), (10, 1), (10, 2), (11, 0), (11, 1), (11, 2),
                  (15, 0), (15, 1), (15, 2), (20, 1), (21, 1)], dtype=np.int64)

_N_ROWS = 16384
_N_COLS = 63
_N_TERMS = len(_CLIP) + len(_ZERO)
_SCALE = 1.0 / (_N_ROWS * _N_TERMS)

_P_COLS = tuple(int((j - 1) * 3 + a) for j, a, s in _CLIP if s > 0)
_N_COLS_NEG = tuple(int((j - 1) * 3 + a) for j, a, s in _CLIP if s < 0)
_Z_COLS = tuple(int((j - 1) * 3 + a) for j, a in _ZERO)

_NW = 32
_ROWS_PER_W = _N_ROWS // _NW
_BLOCKS = _ROWS_PER_W // 16

_mesh = plsc.VectorSubcoreMesh(core_axis_name="c", subcore_axis_name="s")


@functools.partial(
    pl.kernel,
    out_type=jax.ShapeDtypeStruct((_NW * 16,), jnp.float32),
    mesh=_mesh,
    scratch_types=[
        pltpu.VMEM((_ROWS_PER_W, _N_COLS), jnp.float32),
        pltpu.VMEM((16,), jnp.float32),
    ],
    compiler_params=pltpu.CompilerParams(needs_layout_passes=False),
)
def _sc_partial_sums(pose_hbm, out_hbm, x_v, acc_v):
    wid = lax.axis_index("s") * 2 + lax.axis_index("c")
    row0 = wid * _ROWS_PER_W
    pltpu.sync_copy(pose_hbm.at[pl.ds(row0, _ROWS_PER_W)], x_v)

    lane = lax.iota(jnp.int32, 16)
    zero = jnp.zeros((16,), jnp.float32)

    def body(b, accs):
        p0, p1, n0, n1, z0, z1 = accs
        rows = lane + b * 16
        pr = []
        for c in _P_COLS:
            x = plsc.load_gather(x_v, [rows, jnp.full((16,), c, jnp.int32)])
            pr.append(jnp.maximum(x, 0.0))
        nr = []
        for c in _N_COLS_NEG:
            x = plsc.load_gather(x_v, [rows, jnp.full((16,), c, jnp.int32)])
            nr.append(jnp.minimum(x, 0.0))
        zr = []
        for c in _Z_COLS:
            x = plsc.load_gather(x_v, [rows, jnp.full((16,), c, jnp.int32)])
            zr.append(jnp.abs(x))
        p0 = p0 + sum(pr[0::2], zero)
        p1 = p1 + sum(pr[1::2], zero)
        n0 = n0 + sum(nr[0::2], zero)
        n1 = n1 + sum(nr[1::2], zero)
        z0 = z0 + sum(zr[0::2], zero)
        z1 = z1 + sum(zr[1::2], zero)
        return (p0, p1, n0, n1, z0, z1)

    init = (zero,) * 6
    p0, p1, n0, n1, z0, z1 = lax.fori_loop(0, _BLOCKS, body, init)
    acc = ((p0 + p1) - (n0 + n1) + (z0 + z1)) * jnp.float32(_SCALE)
    acc_v[...] = acc
    pltpu.sync_copy(acc_v, out_hbm.at[pl.ds(wid * 16, 16)])


def kernel(pose):
    partials = _sc_partial_sums(pose)
    return jnp.sum(partials)
